# baseline (speedup 1.0000x reference)
import functools

import jax
import jax.numpy as jnp
from jax import lax
from jax.experimental import pallas as pl
from jax.experimental.pallas import tpu as pltpu
from jax.experimental.pallas import tpu_sc as plsc

_GAMMA = 3.0
_C = 128
_L = 16
_NC = 2
_NS = 16
_UNR = 8
_R0 = 3


def _make_dist_kernel(d, p_pad, cpw0, cpw1):
  mesh = plsc.VectorSubcoreMesh(
      core_axis_name="c", subcore_axis_name="s", num_cores=_NC,
      num_subcores=_NS)

  @functools.partial(
      pl.kernel,
      mesh=mesh,
      compiler_params=pltpu.CompilerParams(needs_layout_passes=False),
      out_type=jax.ShapeDtypeStruct((p_pad,), jnp.float32),
      scratch_types=[
          pltpu.VMEM((cpw0, _C), jnp.int32),
          pltpu.VMEM((cpw0, _C), jnp.int32),
          pltpu.VMEM((_C, d), jnp.float32),
          pltpu.VMEM((_C, d), jnp.float32),
          pltpu.VMEM((_C, d), jnp.float32),
          pltpu.VMEM((_C, d), jnp.float32),
          [pltpu.VMEM((_C,), jnp.float32)] * 4,
          pltpu.SemaphoreType.DMA,
          pltpu.SemaphoreType.DMA,
          pltpu.SemaphoreType.DMA,
          pltpu.SemaphoreType.DMA,
          [pltpu.SemaphoreType.DMA] * 4,
      ],
  )
  def dist_kernel(emb, idxl, idxr, out, idxl_v, idxr_v, rl0, rr0, rl1, rr1,
                  dists, sl0, sr0, sl1, sr1, swbs):
    cid = lax.axis_index("c")
    sid = lax.axis_index("s")
    my_cpw = jnp.where(cid == 0, cpw0, cpw1)
    start = jnp.where(cid == 0, sid * cpw0, _NS * cpw0 + sid * cpw1)
    lane = lax.iota(jnp.int32, _L)

    pltpu.async_copy(idxl.at[pl.ds(start, cpw0)], idxl_v, sl0)
    pltpu.async_copy(idxr.at[pl.ds(start, cpw0)], idxr_v, sr0)
    pltpu.make_async_copy(idxl.at[pl.ds(start, cpw0)], idxl_v, sl0).wait()
    pltpu.make_async_copy(idxr.at[pl.ds(start, cpw0)], idxr_v, sr0).wait()

    def fire(t, rl, rr, sl, sr):
      pltpu.async_copy(emb.at[idxl_v.at[t]], rl, sl)
      pltpu.async_copy(emb.at[idxr_v.at[t]], rr, sr)

    def drain(t, rl, rr, sl, sr):
      pltpu.make_async_copy(emb.at[idxl_v.at[t]], rl, sl).wait()
      pltpu.make_async_copy(emb.at[idxr_v.at[t]], rr, sr).wait()

    def compute(t, rl, rr, dist_v):
      for g in range(_C // _L):
        row_idx = lane + (g * _L)

        def col_body(s, acc):
          for u in range(_UNR):
            dcol = s * _UNR + u
            col_idx = (lane + dcol) & (d - 1)
            lv = plsc.load_gather(rl, [row_idx, col_idx])
            rv = plsc.load_gather(rr, [row_idx, col_idx])
            acc = acc + jnp.abs(lv - rv)
          return acc

        acc = lax.fori_loop(0, d // _UNR, col_body,
                            jnp.zeros((_L,), jnp.float32))
        dist_v[pl.ds(g * _L, _L)] = acc

    def wb_fire(t, slot):
      pltpu.async_copy(dists[slot], out.at[pl.ds((start + t) * _C, _C)],
                       swbs[slot])

    def wb_drain(t, slot):
      pltpu.make_async_copy(dists[slot],
                            out.at[pl.ds((start + t) * _C, _C)],
                            swbs[slot]).wait()

    fire(0, rl0, rr0, sl0, sr0)

    def body(tt, carry):
      t = 4 * tt
      bufs = ((rl0, rr0, sl0, sr0), (rl1, rr1, sl1, sr1))
      for j in range(4):
        rl, rr, sl, sr = bufs[j % 2]
        nrl, nrr, nsl, nsr = bufs[(j + 1) % 2]
        if j < 3:
          fire(t + j + 1, nrl, nrr, nsl, nsr)
        drain(t + j, rl, rr, sl, sr)

        @pl.when(tt > 0)
        def _wb_done():
          wb_drain(t + j - 4, j)

        compute(t + j, rl, rr, dists[j])
        wb_fire(t + j, j)
        if j == 3:
          @pl.when(t + 4 < my_cpw)
          def _prefetch():
            fire(t + 4, rl0, rr0, sl0, sr0)

      return carry

    lax.fori_loop(0, my_cpw // 4, body, 0)
    for j in range(4):
      wb_drain(my_cpw - 4 + j, j)

  return dist_kernel


def _hinge(a, b1, b2, t, k):
  steps = 10
  rows = t // steps
  inv = 1.0 / (2.0 * k * t)

  def body(a_ref, b1_ref, b2_ref, o_ref):
    @pl.when(pl.program_id(0) == 0)
    def _init():
      o_ref[0, 0] = 0.0

    dv = a_ref[...] + _GAMMA
    s1 = jnp.sum(jnp.maximum(dv - b1_ref[...], 0.0))
    s2 = jnp.sum(jnp.maximum(dv - b2_ref[...], 0.0))
    o_ref[0, 0] += (s1 + s2) * inv

  out = pl.pallas_call(
      body,
      grid=(steps,),
      in_specs=[
          pl.BlockSpec((rows, 1), lambda i: (i, 0)),
          pl.BlockSpec((rows, k), lambda i: (i, 0)),
          pl.BlockSpec((rows, k), lambda i: (i, 0)),
      ],
      out_specs=pl.BlockSpec((1, 1), lambda i: (0, 0),
                             memory_space=pltpu.SMEM),
      out_shape=jax.ShapeDtypeStruct((1, 1), jnp.float32),
  )(a, b1, b2)
  return out[0, 0]


def kernel(out_emb, ILL, neg_left1, neg_right1, neg_left2, neg_right2):
  n, d = out_emb.shape
  t = ILL.shape[0]
  k = neg_left1.shape[0] // t
  p = t + 2 * t * k
  q = -(-p // (_C * _NS * (_R0 + 1)))
  cpw1 = -(-q // 4) * 4
  cpw0 = _R0 * cpw1
  ct = _NS * (cpw0 + cpw1)
  p_pad = ct * _C
  pad = p_pad - p

  zpad = jnp.zeros((pad,), jnp.int32)
  left = jnp.concatenate([ILL[:, 0], neg_left1, neg_left2, zpad])
  right = jnp.concatenate([ILL[:, 1], neg_right1, neg_right2, zpad])
  ipad = jnp.zeros(((cpw0 - cpw1) * _C,), jnp.int32)
  left = jnp.concatenate([left, ipad]).reshape(-1, _C)
  right = jnp.concatenate([right, ipad]).reshape(-1, _C)

  dist = _make_dist_kernel(d, p_pad, cpw0, cpw1)(out_emb, left, right)

  a = dist[:t].reshape(t, 1)
  b1 = dist[t:t + t * k].reshape(t, k)
  b2 = dist[t + t * k:t + 2 * t * k].reshape(t, k)
  return _hinge(a, b1, b2, t, k)


# speedup vs baseline: 1.4686x; 1.4686x over previous
import functools

import jax
import jax.numpy as jnp
from jax import lax
from jax.experimental import pallas as pl
from jax.experimental.pallas import tpu as pltpu
from jax.experimental.pallas import tpu_sc as plsc

_GAMMA = 3.0
_C = 128
_L = 16
_NC = 2
_NS = 16
_UNR = 8


def _make_dist_kernel(d, p_pad, cpw):
  mesh = plsc.VectorSubcoreMesh(
      core_axis_name="c", subcore_axis_name="s", num_cores=_NC,
      num_subcores=_NS)

  @functools.partial(
      pl.kernel,
      mesh=mesh,
      compiler_params=pltpu.CompilerParams(needs_layout_passes=False),
      out_type=jax.ShapeDtypeStruct((p_pad,), jnp.float32),
      scratch_types=[
          [pltpu.VMEM((_C,), jnp.int32)] * 4,
          [pltpu.VMEM((_C,), jnp.int32)] * 4,
          pltpu.VMEM((_C, d), jnp.float32),
          pltpu.VMEM((_C, d), jnp.float32),
          pltpu.VMEM((_C, d), jnp.float32),
          pltpu.VMEM((_C, d), jnp.float32),
          [pltpu.VMEM((_C,), jnp.float32)] * 4,
          pltpu.SemaphoreType.DMA,
          pltpu.SemaphoreType.DMA,
          pltpu.SemaphoreType.DMA,
          pltpu.SemaphoreType.DMA,
          [pltpu.SemaphoreType.DMA] * 4,
          [pltpu.SemaphoreType.DMA] * 4,
      ],
  )
  def dist_kernel(emb, idxl, idxr, out, ils, irs, rl0, rr0, rl1, rr1,
                  dists, sl0, sr0, sl1, sr1, swbs, sidx):
    cid = lax.axis_index("c")
    sid = lax.axis_index("s")
    start = sid * cpw
    lane = lax.iota(jnp.int32, _L)

    def fire_idx(u, slot):
      pltpu.async_copy(idxl.at[start + u], ils[slot], sidx[slot])
      pltpu.async_copy(idxr.at[start + u], irs[slot], sidx[slot])

    def wait_idx(u, slot):
      pltpu.make_async_copy(idxl.at[start + u], ils[slot],
                            sidx[slot]).wait()
      pltpu.make_async_copy(idxr.at[start + u], irs[slot],
                            sidx[slot]).wait()

    def fire(slot, rl, rr, sl, sr):
      pltpu.async_copy(emb.at[ils[slot]], rl, sl)
      pltpu.async_copy(emb.at[irs[slot]], rr, sr)

    def drain(slot, rl, rr, sl, sr):
      pltpu.make_async_copy(emb.at[ils[slot]], rl, sl).wait()
      pltpu.make_async_copy(emb.at[irs[slot]], rr, sr).wait()

    def compute(rl, rr, dist_v):
      for g in range(_C // _L):
        row_idx = lane + (g * _L)

        def col_body(s, acc):
          for u in range(_UNR):
            dcol = s * _UNR + u
            col_idx = (lane + dcol) & (d - 1)
            lv = plsc.load_gather(rl, [row_idx, col_idx])
            rv = plsc.load_gather(rr, [row_idx, col_idx])
            acc = acc + jnp.abs(lv - rv)
          return acc

        acc = lax.fori_loop(0, d // _UNR, col_body,
                            jnp.zeros((_L,), jnp.float32))
        dist_v[pl.ds(g * _L, _L)] = acc

    def wb_fire(u, slot):
      pltpu.async_copy(dists[slot], out.at[pl.ds((start + u) * _C, _C)],
                       swbs[slot])

    def wb_drain(u, slot):
      pltpu.make_async_copy(dists[slot],
                            out.at[pl.ds((start + u) * _C, _C)],
                            swbs[slot]).wait()

    def body(tt, carry):
      t = 4 * tt
      bufs = ((rl0, rr0, sl0, sr0), (rl1, rr1, sl1, sr1))
      for j in range(4):
        u = t + j
        rl, rr, sl, sr = bufs[j % 2]
        nrl, nrr, nsl, nsr = bufs[(j + 1) % 2]

        @pl.when(u + 1 < cpw)
        def _next_rows():
          wait_idx(u + 1, (j + 1) % 4)
          fire((j + 1) % 4, nrl, nrr, nsl, nsr)

        @pl.when(u + 3 < cpw)
        def _next_idx():
          fire_idx(u + 3, (j + 3) % 4)

        drain(j % 4, rl, rr, sl, sr)

        @pl.when(tt > 0)
        def _wb_done():
          wb_drain(u - 4, j)

        compute(rl, rr, dists[j])
        wb_fire(u, j)

      return carry

    @pl.when(cid == 0)
    def _run():
      fire_idx(0, 0)
      fire_idx(1, 1)
      fire_idx(2, 2)
      wait_idx(0, 0)
      fire(0, rl0, rr0, sl0, sr0)
      lax.fori_loop(0, cpw // 4, body, 0)
      for j in range(4):
        wb_drain(cpw - 4 + j, j)

  return dist_kernel


def _hinge(a, b1, b2, t, k):
  steps = 10
  rows = t // steps
  inv = 1.0 / (2.0 * k * t)

  def body(a_ref, b1_ref, b2_ref, o_ref):
    @pl.when(pl.program_id(0) == 0)
    def _init():
      o_ref[0, 0] = 0.0

    dv = a_ref[...] + _GAMMA
    s1 = jnp.sum(jnp.maximum(dv - b1_ref[...], 0.0))
    s2 = jnp.sum(jnp.maximum(dv - b2_ref[...], 0.0))
    o_ref[0, 0] += (s1 + s2) * inv

  out = pl.pallas_call(
      body,
      grid=(steps,),
      in_specs=[
          pl.BlockSpec((rows, 1), lambda i: (i, 0)),
          pl.BlockSpec((rows, k), lambda i: (i, 0)),
          pl.BlockSpec((rows, k), lambda i: (i, 0)),
      ],
      out_specs=pl.BlockSpec((1, 1), lambda i: (0, 0),
                             memory_space=pltpu.SMEM),
      out_shape=jax.ShapeDtypeStruct((1, 1), jnp.float32),
  )(a, b1, b2)
  return out[0, 0]


def kernel(out_emb, ILL, neg_left1, neg_right1, neg_left2, neg_right2):
  n, d = out_emb.shape
  t = ILL.shape[0]
  k = neg_left1.shape[0] // t
  p = t + 2 * t * k
  cpw = -(-p // (_C * _NS * 4)) * 4
  ct = _NS * cpw
  p_pad = ct * _C
  pad = p_pad - p

  zpad = jnp.zeros((pad,), jnp.int32)
  left = jnp.concatenate([ILL[:, 0], neg_left1, neg_left2, zpad])
  right = jnp.concatenate([ILL[:, 1], neg_right1, neg_right2, zpad])
  left = left.reshape(ct, _C)
  right = right.reshape(ct, _C)

  dist = _make_dist_kernel(d, p_pad, cpw)(out_emb, left, right)

  a = dist[:t].reshape(t, 1)
  b1 = dist[t:t + t * k].reshape(t, k)
  b2 = dist[t + t * k:t + 2 * t * k].reshape(t, k)
  return _hinge(a, b1, b2, t, k)
